# baseline (speedup 1.0000x reference)
import jax
import jax.numpy as jnp
from jax.experimental import pallas as pl
from jax.experimental.pallas import tpu as pltpu
from jax.experimental.pallas import tpu_sc as plsc

_WINDOW = 128


def _sc_gather(data, idx_main, idx_tail, main, tail, d):
    total = main + tail

    @pl.kernel(
        out_type=jax.ShapeDtypeStruct((total, d), data.dtype),
        mesh=plsc.VectorSubcoreMesh(
            core_axis_name="core", subcore_axis_name="subcore"
        ),
    )
    def gather_kernel(data_hbm, idx_main_hbm, idx_tail_hbm, out_hbm):
        def body(idx_vmem, out_vmem):
            pltpu.sync_copy(data_hbm.at[idx_vmem.at[0]], out_vmem)

        pltpu.emit_pipeline(
            body,
            grid=(main // _WINDOW,),
            in_specs=[
                pl.BlockSpec((1, _WINDOW), index_map=lambda i: (0, i))
            ],
            out_specs=[
                pl.BlockSpec((_WINDOW, d), index_map=lambda i: (i, 0))
            ],
            core_axis_name=("core", "subcore"),
            dimension_semantics=(pltpu.PARALLEL,),
        )(idx_main_hbm, out_hbm)

        if tail:
            pltpu.emit_pipeline(
                body,
                grid=(1,),
                in_specs=[
                    pl.BlockSpec((1, tail), index_map=lambda i: (0, 0))
                ],
                out_specs=[
                    pl.BlockSpec(
                        (tail, d), index_map=lambda i: (main // tail, 0)
                    )
                ],
                core_axis_name=("core", "subcore"),
                dimension_semantics=(pltpu.PARALLEL,),
            )(idx_tail_hbm, out_hbm)

    return gather_kernel(data, idx_main, idx_tail)


def kernel(data, indices):
    n, d = data.shape
    k = indices.shape[1]
    total = n * (k + 1)
    neigh = indices.reshape(n, k).astype(jnp.int32)
    self_idx = jnp.arange(n, dtype=jnp.int32)[:, None]
    idx_flat = jnp.concatenate([self_idx, neigh], axis=1).reshape(total)
    main = (total // _WINDOW) * _WINDOW
    tail = total - main
    idx_main = idx_flat[:main].reshape(1, main)
    idx_tail = idx_flat[main:].reshape(1, tail) if tail else idx_flat[:1].reshape(1, 1)
    out = _sc_gather(data, idx_main, idx_tail, main, tail, d)
    return out.reshape(n, k + 1, d)


# speedup vs baseline: 1.4546x; 1.4546x over previous
import jax
import jax.numpy as jnp
from jax.experimental import pallas as pl
from jax.experimental.pallas import tpu as pltpu
from jax.experimental.pallas import tpu_sc as plsc

_NB = 8


def _sc_gather(data, idx_a, idx_b, n, k, d):
    @pl.kernel(
        out_type=jax.ShapeDtypeStruct((n, k + 1, d), data.dtype),
        mesh=plsc.VectorSubcoreMesh(
            core_axis_name="core", subcore_axis_name="subcore"
        ),
    )
    def gather_kernel(data_hbm, idx_a_hbm, idx_b_hbm, out_hbm):
        def body(idx_a_vmem, idx_b_vmem, out_vmem):
            for j in range(_NB):
                pltpu.sync_copy(
                    data_hbm.at[idx_a_vmem.at[j]],
                    out_vmem.at[j, pl.ds(0, k)],
                )
                pltpu.sync_copy(
                    data_hbm.at[idx_b_vmem.at[j]],
                    out_vmem.at[j, pl.ds(k, 1)],
                )

        pltpu.emit_pipeline(
            body,
            grid=(n // _NB,),
            in_specs=[
                pl.BlockSpec((_NB, k), index_map=lambda i: (i, 0)),
                pl.BlockSpec((_NB, 1), index_map=lambda i: (i, 0)),
            ],
            out_specs=[
                pl.BlockSpec(
                    (_NB, k + 1, d), index_map=lambda i: (i, 0, 0)
                )
            ],
            core_axis_name=("core", "subcore"),
            dimension_semantics=(pltpu.PARALLEL,),
        )(idx_a_hbm, idx_b_hbm, out_hbm)

    return gather_kernel(data, idx_a, idx_b)


def kernel(data, indices):
    n, d = data.shape
    k = indices.shape[1]
    neigh = indices.reshape(n, k).astype(jnp.int32)
    self_idx = jnp.arange(n, dtype=jnp.int32)[:, None]
    idx_a = jnp.concatenate([self_idx, neigh[:, : k - 1]], axis=1)
    idx_b = neigh[:, k - 1 :]
    return _sc_gather(data, idx_a, idx_b, n, k, d)


# speedup vs baseline: 2.9456x; 2.0251x over previous
import jax
import jax.numpy as jnp
from jax.experimental import pallas as pl
from jax.experimental.pallas import tpu as pltpu
from jax.experimental.pallas import tpu_sc as plsc

_NB = 8


def _sc_gather(data, idx_a, idx_b, n, k, d):
    @pl.kernel(
        out_type=jax.ShapeDtypeStruct((n, k + 1, d), data.dtype),
        mesh=plsc.VectorSubcoreMesh(
            core_axis_name="core", subcore_axis_name="subcore"
        ),
        scratch_types=[pltpu.SemaphoreType.DMA((2 * _NB,))],
    )
    def gather_kernel(data_hbm, idx_a_hbm, idx_b_hbm, out_hbm, sems):
        def body(idx_a_vmem, idx_b_vmem, out_vmem):
            copies = []
            for j in range(_NB):
                copies.append(
                    pltpu.async_copy(
                        data_hbm.at[idx_a_vmem.at[j]],
                        out_vmem.at[j, pl.ds(0, k)],
                        sems.at[2 * j],
                    )
                )
                copies.append(
                    pltpu.async_copy(
                        data_hbm.at[idx_b_vmem.at[j]],
                        out_vmem.at[j, pl.ds(k, 1)],
                        sems.at[2 * j + 1],
                    )
                )
            for c in copies:
                c.wait()

        pltpu.emit_pipeline(
            body,
            grid=(n // _NB,),
            in_specs=[
                pl.BlockSpec((_NB, k), index_map=lambda i: (i, 0)),
                pl.BlockSpec((_NB, 1), index_map=lambda i: (i, 0)),
            ],
            out_specs=[
                pl.BlockSpec(
                    (_NB, k + 1, d), index_map=lambda i: (i, 0, 0)
                )
            ],
            core_axis_name=("core", "subcore"),
            dimension_semantics=(pltpu.PARALLEL,),
        )(idx_a_hbm, idx_b_hbm, out_hbm)

    return gather_kernel(data, idx_a, idx_b)


def kernel(data, indices):
    n, d = data.shape
    k = indices.shape[1]
    neigh = indices.reshape(n, k).astype(jnp.int32)
    self_idx = jnp.arange(n, dtype=jnp.int32)[:, None]
    idx_a = jnp.concatenate([self_idx, neigh[:, : k - 1]], axis=1)
    idx_b = neigh[:, k - 1 :]
    return _sc_gather(data, idx_a, idx_b, n, k, d)
